# Initial kernel scaffold; baseline (speedup 1.0000x reference)
import jax
import jax.numpy as jnp
from jax.experimental import pallas as pl


def kernel(A1, A2, Wq, bq, Wk, bk, Wv, bv):
    raise NotImplementedError("write your pallas kernel here")


# speedup vs baseline: 1.8303x; 1.8303x over previous
import jax
import jax.numpy as jnp
from jax import lax
from jax.experimental import pallas as pl
from jax.experimental.pallas import tpu as pltpu
from jax.experimental.pallas import tpu_sc as plsc

_B = 64
_N = 32768
_K = 50
_L = 16
_NV = _N // _L
_CAP = 512
_CAPV = _CAP // _L
_KP = 64
_KPV = _KP // _L
_WROW = _K * _KP
_NEG = -3.0e38


def _rnd_bf16(v):
  u = plsc.bitcast(v, jnp.uint32)
  r = (u + jnp.uint32(0x7FFF) + ((u >> jnp.uint32(16)) & jnp.uint32(1))) \
      & jnp.uint32(0xFFFF0000)
  return plsc.bitcast(r, jnp.float32)


def _exp_f32(x):
  xc = jnp.maximum(x, jnp.float32(-87.0))
  t = xc * jnp.float32(1.4426950408889634)
  k = lax.convert_element_type(t - jnp.float32(0.5), jnp.int32)
  r = t - lax.convert_element_type(k, jnp.float32)
  c1 = jnp.float32(0.6931471805599453)
  c2 = jnp.float32(0.2402265069591007)
  c3 = jnp.float32(0.05550410866482158)
  c4 = jnp.float32(0.009618129107628477)
  c5 = jnp.float32(0.0013333558146428443)
  p = jnp.float32(1.0) + r * (c1 + r * (c2 + r * (c3 + r * (c4 + r * c5))))
  scale = plsc.bitcast((k + jnp.int32(127)) << jnp.int32(23), jnp.float32)
  return p * scale


def _key_desc(v):
  u = plsc.bitcast(v, jnp.uint32)
  m = jnp.where(u >= jnp.uint32(0x80000000), ~u, u | jnp.uint32(0x80000000))
  return ~m


def _crossing(hist_ref, target):
  def body(g, c):
    cum, bstar, cbef = c
    s = jnp.sum(hist_ref[pl.ds(g * _L, _L)])
    newcum = cum + s
    hit = jnp.logical_and(cum < target, newcum >= target)
    bstar = jnp.where(hit, g, bstar)
    cbef = jnp.where(hit, cum, cbef)
    return newcum, bstar, cbef
  _, bstar, cbef = lax.fori_loop(
      0, 256, body, (jnp.int32(0), jnp.int32(0), jnp.int32(0)))
  return bstar, cbef


def _zero_hist(hist_ref):
  z = jnp.zeros((_L,), jnp.int32)
  def body(g, _):
    hist_ref[pl.ds(g * _L, _L)] = z
    return 0
  lax.fori_loop(0, 256, body, 0)


def _rot(x, k, lane):
  if k == 0:
    return x
  perm = (lane + jnp.int32(k)) & jnp.int32(_L - 1)
  return jnp.take_along_axis(x, perm, axis=0)


def _topk_row(row_ref, hist_ref, cand_v_ref, cand_i_ref, sv_ref, si_ref):
  lane = lax.iota(jnp.int32, _L)
  ones = jnp.ones((_L,), jnp.int32)

  _zero_hist(hist_ref)
  def p1(i, _):
    d = _key_desc(row_ref[pl.ds(i * _L, _L)])
    b1 = lax.convert_element_type(d >> jnp.uint32(24), jnp.int32)
    plsc.addupdate_scatter(hist_ref, [b1 * _L + lane], ones)
    return 0
  lax.fori_loop(0, _NV, p1, 0)
  b1star, c1 = _crossing(hist_ref, jnp.int32(_K))

  _zero_hist(hist_ref)
  b1star_u = lax.convert_element_type(b1star, jnp.uint32)
  def p2(i, _):
    d = _key_desc(row_ref[pl.ds(i * _L, _L)])
    msk = (d >> jnp.uint32(24)) == b1star_u
    b2 = lax.convert_element_type((d >> jnp.uint32(16)) & jnp.uint32(0xFF),
                                  jnp.int32)
    plsc.addupdate_scatter(hist_ref, [b2 * _L + lane], ones, mask=msk)
    return 0
  lax.fori_loop(0, _NV, p2, 0)
  b2star, _ = _crossing(hist_ref, jnp.int32(_K) - c1)
  t16 = (b1star_u << jnp.uint32(8)) | lax.convert_element_type(
      b2star, jnp.uint32)

  ninf = jnp.full((_L,), _NEG, jnp.float32)
  bigi = jnp.full((_L,), jnp.int32(_N), jnp.int32)
  def pf(g, _):
    cand_v_ref[pl.ds(g * _L, _L)] = ninf
    cand_i_ref[pl.ds(g * _L, _L)] = bigi
    return 0
  lax.fori_loop(0, _CAPV, pf, 0)

  def p3(i, cnt):
    v = row_ref[pl.ds(i * _L, _L)]
    d = _key_desc(v)
    msk = (d >> jnp.uint32(16)) <= t16
    mi = msk.astype(jnp.int32)
    pos = jnp.minimum(cnt + plsc.cumsum(mi) - mi, jnp.int32(_CAP - 1))
    plsc.store_scatter(cand_v_ref, [pos], v, mask=msk)
    plsc.store_scatter(cand_i_ref, [pos], i * _L + lane, mask=msk)
    return jnp.minimum(cnt + jnp.sum(mi), jnp.int32(_CAP))
  cnt = lax.fori_loop(0, _NV, p3, jnp.int32(0))

  zf = jnp.zeros((_L,), jnp.float32)
  zi = jnp.zeros((_L,), jnp.int32)
  for g in range(_KPV):
    sv_ref[pl.ds(g * _L, _L)] = zf
    si_ref[pl.ds(g * _L, _L)] = zi

  nv = (cnt + _L - 1) // _L
  def rank_x(x, _):
    vx = cand_v_ref[pl.ds(x * _L, _L)]
    ix = cand_i_ref[pl.ds(x * _L, _L)]
    def rank_y(y, r):
      vy = cand_v_ref[pl.ds(y * _L, _L)]
      iy = cand_i_ref[pl.ds(y * _L, _L)]
      for k in range(_L):
        vyk = _rot(vy, k, lane)
        iyk = _rot(iy, k, lane)
        gt = jnp.logical_or(
            vyk > vx, jnp.logical_and(vyk == vx, iyk < ix))
        r = r + gt.astype(jnp.int32)
      return r
    rank = lax.fori_loop(0, nv, rank_y, jnp.zeros((_L,), jnp.int32))
    sel = rank < _K
    plsc.store_scatter(sv_ref, [rank], vx, mask=sel)
    plsc.store_scatter(si_ref, [rank], ix, mask=sel)
    return 0
  lax.fori_loop(0, nv, rank_x, 0)


def _sc_kernel(a1_hbm, a2_hbm, par_hbm, upd_hbm, wts_hbm,
               a1row, a2row, hist, cand_v, cand_i,
               sv1, si1, sv2, si2, wbuf, kbt, qb, uvb, parbuf, sem):
  wid = lax.axis_index("s") * 2 + lax.axis_index("c")
  pltpu.sync_copy(par_hbm, parbuf)
  wqv = parbuf[pl.ds(0 * _L, _L)]
  bqv = parbuf[pl.ds(1 * _L, _L)]
  wkv = parbuf[pl.ds(2 * _L, _L)]
  bkv = parbuf[pl.ds(3 * _L, _L)]
  wvv = parbuf[pl.ds(4 * _L, _L)]
  bvv = parbuf[pl.ds(5 * _L, _L)]
  lane = lax.iota(jnp.int32, _L)

  def row_body(r, _):
    b = wid * 2 + r
    pltpu.sync_copy(a1_hbm.at[b], a1row)
    pltpu.sync_copy(a2_hbm.at[b], a2row)

    _topk_row(a1row, hist, cand_v, cand_i, sv1, si1)
    _topk_row(a2row, hist, cand_v, cand_i, sv2, si2)

    valids = [(g * _L + lane) < _K for g in range(_KPV)]
    z16 = jnp.zeros((_L,), jnp.float32)
    for h in range(_L):
      for g in range(_KPV):
        kbt[pl.ds(h * _KP + g * _L, _L)] = z16
    kv_g = [sv2[pl.ds(g * _L, _L)] for g in range(_KPV)]
    mvv = [jnp.zeros((_L,), jnp.float32) for _ in range(_KPV)]
    for j in range(_K):
      kj = kv_g[j // _L][j % _L]
      kb = _rnd_bf16(kj * wkv + bkv)
      plsc.store_scatter(kbt, [lane * _KP + jnp.int32(j)], kb)
      vb = _rnd_bf16(kj * wvv + bvv)
      mvj = jnp.sum(vb) * jnp.float32(0.0625)
      mvv[j // _L] = jnp.where(lane == jnp.int32(j % _L), mvj, mvv[j // _L])
    qv_g = [sv1[pl.ds(g * _L, _L)] for g in range(_KPV)]
    for i in range(_K):
      qi = qv_g[i // _L][i % _L]
      qb[pl.ds(i * _L, _L)] = _rnd_bf16(qi * wqv + bqv)

    def att(i, _):
      qvec = qb[pl.ds(i * _L, _L)]
      qs = [qvec[h] for h in range(_L)]
      mx = _NEG
      scs = []
      for g in range(_KPV):
        acc = jnp.zeros((_L,), jnp.float32)
        for h in range(_L):
          acc = acc + qs[h] * kbt[pl.ds(h * _KP + g * _L, _L)]
        sc = jnp.where(valids[g], acc, _NEG)
        scs.append(sc)
        mx = jnp.maximum(mx, jnp.max(sc))
      tot = jnp.float32(0.0)
      es = []
      for g in range(_KPV):
        arg = jnp.where(valids[g], scs[g] - mx, jnp.float32(-87.0))
        e = jnp.where(valids[g], _exp_f32(arg), 0.0)
        es.append(e)
        tot = tot + jnp.sum(e)
      invv = jnp.ones((_L,), jnp.float32) / jnp.full((_L,), tot)
      dot = jnp.float32(0.0)
      for g in range(_KPV):
        w = es[g] * invv
        wbuf[pl.ds(i * _KP + g * _L, _L)] = w
        dot = dot + jnp.sum(_rnd_bf16(w) * mvv[g])
      plsc.store_scatter(uvb, [jnp.full((_L,), i, jnp.int32)],
                         jnp.full((_L,), dot, jnp.float32),
                         mask=(lane == 0))
      return 0
    lax.fori_loop(0, _K, att, 0)

    z = jnp.zeros((_L,), jnp.float32)
    def zb(g, _):
      a1row[pl.ds(g * _L, _L)] = z
      return 0
    lax.fori_loop(0, _NV, zb, 0)
    for g in range(_KPV):
      idx = si2[pl.ds(g * _L, _L)]
      plsc.store_scatter(a1row, [idx], uvb[pl.ds(g * _L, _L)],
                         mask=valids[g])

    pltpu.sync_copy(a1row, upd_hbm.at[b])
    pltpu.sync_copy(wbuf, wts_hbm.at[b])
    return 0

  lax.fori_loop(0, 2, row_body, 0)


@jax.jit
def kernel(A1, A2, Wq, bq, Wk, bk, Wv, bv):
  par = jnp.concatenate([Wq[:, 0], bq, Wk[:, 0], bk, Wv[:, 0], bv])

  mesh = plsc.VectorSubcoreMesh(core_axis_name="c", subcore_axis_name="s",
                                num_cores=2, num_subcores=16)
  f = pl.kernel(
      _sc_kernel,
      out_type=[
          jax.ShapeDtypeStruct((_B, _N), jnp.float32),
          jax.ShapeDtypeStruct((_B, _WROW), jnp.float32),
      ],
      mesh=mesh,
      compiler_params=pltpu.CompilerParams(needs_layout_passes=False),
      scratch_types=[
          pltpu.VMEM((_N,), jnp.float32),
          pltpu.VMEM((_N,), jnp.float32),
          pltpu.VMEM((256 * _L,), jnp.int32),
          pltpu.VMEM((_CAP,), jnp.float32),
          pltpu.VMEM((_CAP,), jnp.int32),
          pltpu.VMEM((_KP,), jnp.float32),
          pltpu.VMEM((_KP,), jnp.int32),
          pltpu.VMEM((_KP,), jnp.float32),
          pltpu.VMEM((_KP,), jnp.int32),
          pltpu.VMEM((_WROW,), jnp.float32),
          pltpu.VMEM((_L * _KP,), jnp.float32),
          pltpu.VMEM((_K * _L,), jnp.float32),
          pltpu.VMEM((_KP,), jnp.float32),
          pltpu.VMEM((6 * _L,), jnp.float32),
          pltpu.SemaphoreType.DMA,
      ],
  )
  upd, wflat = f(A1, A2, par)
  weights = wflat.reshape(_B, _K, _KP)[:, :, :_K]
  return upd, weights


# speedup vs baseline: 1.9444x; 1.0623x over previous
import jax
import jax.numpy as jnp
from jax import lax
from jax.experimental import pallas as pl
from jax.experimental.pallas import tpu as pltpu
from jax.experimental.pallas import tpu_sc as plsc

_B = 64
_N = 32768
_K = 50
_L = 16
_NV = _N // _L
_CAP = 512
_CAPV = _CAP // _L
_KP = 64
_KPV = _KP // _L
_WROW = _K * _KP
_NEG = -3.0e38


def _rnd_bf16(v):
  u = plsc.bitcast(v, jnp.uint32)
  r = (u + jnp.uint32(0x7FFF) + ((u >> jnp.uint32(16)) & jnp.uint32(1))) \
      & jnp.uint32(0xFFFF0000)
  return plsc.bitcast(r, jnp.float32)


def _exp_f32(x):
  xc = jnp.maximum(x, jnp.float32(-87.0))
  t = xc * jnp.float32(1.4426950408889634)
  k = lax.convert_element_type(t - jnp.float32(0.5), jnp.int32)
  r = t - lax.convert_element_type(k, jnp.float32)
  c1 = jnp.float32(0.6931471805599453)
  c2 = jnp.float32(0.2402265069591007)
  c3 = jnp.float32(0.05550410866482158)
  c4 = jnp.float32(0.009618129107628477)
  c5 = jnp.float32(0.0013333558146428443)
  p = jnp.float32(1.0) + r * (c1 + r * (c2 + r * (c3 + r * (c4 + r * c5))))
  scale = plsc.bitcast((k + jnp.int32(127)) << jnp.int32(23), jnp.float32)
  return p * scale


def _key_desc(v):
  u = plsc.bitcast(v, jnp.uint32)
  m = jnp.where(u >= jnp.uint32(0x80000000), ~u, u | jnp.uint32(0x80000000))
  return ~m


def _crossing(hist_ref, target):
  def body(g, c):
    cum, bstar, cbef = c
    s = jnp.sum(hist_ref[pl.ds(g * _L, _L)])
    newcum = cum + s
    hit = jnp.logical_and(cum < target, newcum >= target)
    bstar = jnp.where(hit, g, bstar)
    cbef = jnp.where(hit, cum, cbef)
    return newcum, bstar, cbef
  _, bstar, cbef = lax.fori_loop(
      0, 256, body, (jnp.int32(0), jnp.int32(0), jnp.int32(0)))
  return bstar, cbef


def _zero_hist(hist_ref):
  z = jnp.zeros((_L,), jnp.int32)
  def body(g, _):
    hist_ref[pl.ds(g * _L, _L)] = z
    return 0
  lax.fori_loop(0, 256, body, 0)


def _rot(x, k, lane):
  if k == 0:
    return x
  perm = (lane + jnp.int32(k)) & jnp.int32(_L - 1)
  return jnp.take_along_axis(x, perm, axis=0)


def _topk_row(row_ref, hist_ref, cand_v_ref, cand_i_ref, sv_ref, si_ref):
  lane = lax.iota(jnp.int32, _L)
  ones = jnp.ones((_L,), jnp.int32)

  _zero_hist(hist_ref)
  def p1(i, _):
    for u in range(4):
      d = _key_desc(row_ref[pl.ds((i * 4 + u) * _L, _L)])
      b1 = lax.convert_element_type(d >> jnp.uint32(24), jnp.int32)
      plsc.addupdate_scatter(hist_ref, [b1 * _L + lane], ones)
    return 0
  lax.fori_loop(0, _NV // 4, p1, 0)
  b1star, c1 = _crossing(hist_ref, jnp.int32(_K))

  _zero_hist(hist_ref)
  b1star_u = lax.convert_element_type(b1star, jnp.uint32)
  def p2(i, _):
    for u in range(4):
      d = _key_desc(row_ref[pl.ds((i * 4 + u) * _L, _L)])
      msk = (d >> jnp.uint32(24)) == b1star_u
      b2 = lax.convert_element_type((d >> jnp.uint32(16)) & jnp.uint32(0xFF),
                                    jnp.int32)
      plsc.addupdate_scatter(hist_ref, [b2 * _L + lane], ones, mask=msk)
    return 0
  lax.fori_loop(0, _NV // 4, p2, 0)
  b2star, _ = _crossing(hist_ref, jnp.int32(_K) - c1)
  t16 = (b1star_u << jnp.uint32(8)) | lax.convert_element_type(
      b2star, jnp.uint32)

  ninf = jnp.full((_L,), _NEG, jnp.float32)
  bigi = jnp.full((_L,), jnp.int32(_N), jnp.int32)
  def pf(g, _):
    cand_v_ref[pl.ds(g * _L, _L)] = ninf
    cand_i_ref[pl.ds(g * _L, _L)] = bigi
    return 0
  lax.fori_loop(0, _CAPV, pf, 0)

  def p3(i, cnt):
    for u in range(4):
      iu = i * 4 + u
      v = row_ref[pl.ds(iu * _L, _L)]
      d = _key_desc(v)
      msk = (d >> jnp.uint32(16)) <= t16
      mi = msk.astype(jnp.int32)
      pos = jnp.minimum(cnt + plsc.cumsum(mi) - mi, jnp.int32(_CAP - 1))
      plsc.store_scatter(cand_v_ref, [pos], v, mask=msk)
      plsc.store_scatter(cand_i_ref, [pos], iu * _L + lane, mask=msk)
      cnt = jnp.minimum(cnt + jnp.sum(mi), jnp.int32(_CAP))
    return cnt
  cnt = lax.fori_loop(0, _NV // 4, p3, jnp.int32(0))

  zf = jnp.zeros((_L,), jnp.float32)
  zi = jnp.zeros((_L,), jnp.int32)
  for g in range(_KPV):
    sv_ref[pl.ds(g * _L, _L)] = zf
    si_ref[pl.ds(g * _L, _L)] = zi

  nv = (cnt + _L - 1) // _L
  def rank_x(x, _):
    vx = cand_v_ref[pl.ds(x * _L, _L)]
    ix = cand_i_ref[pl.ds(x * _L, _L)]
    def rank_y(y, r):
      vy = cand_v_ref[pl.ds(y * _L, _L)]
      iy = cand_i_ref[pl.ds(y * _L, _L)]
      for k in range(_L):
        vyk = _rot(vy, k, lane)
        iyk = _rot(iy, k, lane)
        gt = jnp.logical_or(
            vyk > vx, jnp.logical_and(vyk == vx, iyk < ix))
        r = r + gt.astype(jnp.int32)
      return r
    rank = lax.fori_loop(0, nv, rank_y, jnp.zeros((_L,), jnp.int32))
    sel = rank < _K
    plsc.store_scatter(sv_ref, [rank], vx, mask=sel)
    plsc.store_scatter(si_ref, [rank], ix, mask=sel)
    return 0
  lax.fori_loop(0, nv, rank_x, 0)


def _sc_kernel(a1_hbm, a2_hbm, par_hbm, upd_hbm, wts_hbm,
               a1row, a2row, hist, cand_v, cand_i,
               sv1, si1, sv2, si2, wbuf, kbt, qb, uvb, parbuf, sem):
  wid = lax.axis_index("s") * 2 + lax.axis_index("c")
  pltpu.sync_copy(par_hbm, parbuf)
  wqv = parbuf[pl.ds(0 * _L, _L)]
  bqv = parbuf[pl.ds(1 * _L, _L)]
  wkv = parbuf[pl.ds(2 * _L, _L)]
  bkv = parbuf[pl.ds(3 * _L, _L)]
  wvv = parbuf[pl.ds(4 * _L, _L)]
  bvv = parbuf[pl.ds(5 * _L, _L)]
  lane = lax.iota(jnp.int32, _L)

  def row_body(r, _):
    b = wid * 2 + r
    pltpu.sync_copy(a1_hbm.at[b], a1row)
    pltpu.sync_copy(a2_hbm.at[b], a2row)

    _topk_row(a1row, hist, cand_v, cand_i, sv1, si1)
    _topk_row(a2row, hist, cand_v, cand_i, sv2, si2)

    valids = [(g * _L + lane) < _K for g in range(_KPV)]
    z16 = jnp.zeros((_L,), jnp.float32)
    for h in range(_L):
      for g in range(_KPV):
        kbt[pl.ds(h * _KP + g * _L, _L)] = z16
    kv_g = [sv2[pl.ds(g * _L, _L)] for g in range(_KPV)]
    mvv = [jnp.zeros((_L,), jnp.float32) for _ in range(_KPV)]
    for j in range(_K):
      kj = kv_g[j // _L][j % _L]
      kb = _rnd_bf16(kj * wkv + bkv)
      plsc.store_scatter(kbt, [lane * _KP + jnp.int32(j)], kb)
      vb = _rnd_bf16(kj * wvv + bvv)
      mvj = jnp.sum(vb) * jnp.float32(0.0625)
      mvv[j // _L] = jnp.where(lane == jnp.int32(j % _L), mvj, mvv[j // _L])
    qv_g = [sv1[pl.ds(g * _L, _L)] for g in range(_KPV)]
    for i in range(_K):
      qi = qv_g[i // _L][i % _L]
      qb[pl.ds(i * _L, _L)] = _rnd_bf16(qi * wqv + bqv)

    def att(i, _):
      qvec = qb[pl.ds(i * _L, _L)]
      qs = [qvec[h] for h in range(_L)]
      mx = _NEG
      scs = []
      for g in range(_KPV):
        acc = jnp.zeros((_L,), jnp.float32)
        for h in range(_L):
          acc = acc + qs[h] * kbt[pl.ds(h * _KP + g * _L, _L)]
        sc = jnp.where(valids[g], acc, _NEG)
        scs.append(sc)
        mx = jnp.maximum(mx, jnp.max(sc))
      tot = jnp.float32(0.0)
      es = []
      for g in range(_KPV):
        arg = jnp.where(valids[g], scs[g] - mx, jnp.float32(-87.0))
        e = jnp.where(valids[g], _exp_f32(arg), 0.0)
        es.append(e)
        tot = tot + jnp.sum(e)
      invv = jnp.ones((_L,), jnp.float32) / jnp.full((_L,), tot)
      dot = jnp.float32(0.0)
      for g in range(_KPV):
        w = es[g] * invv
        wbuf[pl.ds(i * _KP + g * _L, _L)] = w
        dot = dot + jnp.sum(_rnd_bf16(w) * mvv[g])
      plsc.store_scatter(uvb, [jnp.full((_L,), i, jnp.int32)],
                         jnp.full((_L,), dot, jnp.float32),
                         mask=(lane == 0))
      return 0
    lax.fori_loop(0, _K, att, 0)

    z = jnp.zeros((_L,), jnp.float32)
    def zb(g, _):
      for u in range(8):
        a1row[pl.ds((g * 8 + u) * _L, _L)] = z
      return 0
    lax.fori_loop(0, _NV // 8, zb, 0)
    for g in range(_KPV):
      idx = si2[pl.ds(g * _L, _L)]
      plsc.store_scatter(a1row, [idx], uvb[pl.ds(g * _L, _L)],
                         mask=valids[g])

    pltpu.sync_copy(a1row, upd_hbm.at[b])
    pltpu.sync_copy(wbuf, wts_hbm.at[b])
    return 0

  lax.fori_loop(0, 2, row_body, 0)


@jax.jit
def kernel(A1, A2, Wq, bq, Wk, bk, Wv, bv):
  par = jnp.concatenate([Wq[:, 0], bq, Wk[:, 0], bk, Wv[:, 0], bv])

  mesh = plsc.VectorSubcoreMesh(core_axis_name="c", subcore_axis_name="s",
                                num_cores=2, num_subcores=16)
  f = pl.kernel(
      _sc_kernel,
      out_type=[
          jax.ShapeDtypeStruct((_B, _N), jnp.float32),
          jax.ShapeDtypeStruct((_B, _WROW), jnp.float32),
      ],
      mesh=mesh,
      compiler_params=pltpu.CompilerParams(needs_layout_passes=False),
      scratch_types=[
          pltpu.VMEM((_N,), jnp.float32),
          pltpu.VMEM((_N,), jnp.float32),
          pltpu.VMEM((256 * _L,), jnp.int32),
          pltpu.VMEM((_CAP,), jnp.float32),
          pltpu.VMEM((_CAP,), jnp.int32),
          pltpu.VMEM((_KP,), jnp.float32),
          pltpu.VMEM((_KP,), jnp.int32),
          pltpu.VMEM((_KP,), jnp.float32),
          pltpu.VMEM((_KP,), jnp.int32),
          pltpu.VMEM((_WROW,), jnp.float32),
          pltpu.VMEM((_L * _KP,), jnp.float32),
          pltpu.VMEM((_K * _L,), jnp.float32),
          pltpu.VMEM((_KP,), jnp.float32),
          pltpu.VMEM((6 * _L,), jnp.float32),
          pltpu.SemaphoreType.DMA,
      ],
  )
  upd, wflat = f(A1, A2, par)
  weights = wflat.reshape(_B, _K, _KP)[:, :, :_K]
  return upd, weights
